# Initial kernel scaffold; baseline (speedup 1.0000x reference)
import jax
import jax.numpy as jnp
from jax.experimental import pallas as pl


def kernel(token_ids, weight):
    raise NotImplementedError("write your pallas kernel here")


# speedup vs baseline: 1.0939x; 1.0939x over previous
import functools

import jax
import jax.numpy as jnp
from jax import lax
from jax.experimental import pallas as pl
from jax.experimental.pallas import tpu as pltpu
from jax.experimental.pallas import tpu_sc as plsc

_NUM_ROWS = 16384 * 50
_DIM = 32
_NW = 32
_PER_W = _NUM_ROWS // _NW
_IDX_MINOR = 128
_C = 1024
_G = _C // _IDX_MINOR
_NCHUNK = _PER_W // _C

_mesh = plsc.VectorSubcoreMesh(core_axis_name="c", subcore_axis_name="s")


@functools.partial(
    pl.kernel,
    mesh=_mesh,
    out_type=jax.ShapeDtypeStruct((_NUM_ROWS, _DIM), jnp.float32),
    scratch_types=[
        pltpu.VMEM((_G, _IDX_MINOR), jnp.int32),
        pltpu.VMEM((_C, _DIM), jnp.float32),
        pltpu.SemaphoreType.DMA,
    ],
    compiler_params=pltpu.CompilerParams(use_tc_tiling_on_sc=False),
)
def _emb_gather(w_hbm, idx_hbm, out_hbm, idx_v, rows_v, sem):
    wid = lax.axis_index("s") * 2 + lax.axis_index("c")
    base = wid * _PER_W

    def chunk(i, carry):
        row0 = base + i * _C
        g0 = pl.multiple_of(row0 // _IDX_MINOR, 8)
        pltpu.sync_copy(idx_hbm.at[pl.ds(g0, _G)], idx_v)
        handles = []
        for j in range(_G):
            handles.append(pltpu.async_copy(
                w_hbm.at[idx_v.at[j]],
                rows_v.at[pl.ds(j * _IDX_MINOR, _IDX_MINOR)],
                sem))
        for h in handles:
            h.wait()
        pltpu.sync_copy(rows_v, out_hbm.at[pl.ds(row0, _C)])
        return carry

    lax.fori_loop(0, _NCHUNK, chunk, 0)


def kernel(token_ids, weight):
    idx = token_ids.astype(jnp.int32).reshape(_NUM_ROWS // _IDX_MINOR, _IDX_MINOR)
    out = _emb_gather(weight, idx)
    return out.reshape(token_ids.shape + (_DIM,))


# speedup vs baseline: 1.1099x; 1.0146x over previous
import functools

import jax
import jax.numpy as jnp
from jax import lax
from jax.experimental import pallas as pl
from jax.experimental.pallas import tpu as pltpu
from jax.experimental.pallas import tpu_sc as plsc

_NUM_ROWS = 16384 * 50
_DIM = 32
_NW = 32
_PER_W = _NUM_ROWS // _NW
_IDX_MINOR = 128
_C = 1024
_G = _C // _IDX_MINOR
_NCHUNK = _PER_W // _C
_GROWS = _PER_W // _IDX_MINOR

_mesh = plsc.VectorSubcoreMesh(core_axis_name="c", subcore_axis_name="s")


@functools.partial(
    pl.kernel,
    mesh=_mesh,
    out_type=jax.ShapeDtypeStruct((_NUM_ROWS, _DIM), jnp.float32),
    scratch_types=[
        pltpu.VMEM((_GROWS, _IDX_MINOR), jnp.int32),
        pltpu.VMEM((2, _C, _DIM), jnp.float32),
        pltpu.SemaphoreType.DMA,
        pltpu.SemaphoreType.DMA,
    ],
    compiler_params=pltpu.CompilerParams(use_tc_tiling_on_sc=False),
)
def _emb_gather(w_hbm, idx_hbm, out_hbm, idx_v, rows_v, gsem, ssem):
    wid = lax.axis_index("s") * 2 + lax.axis_index("c")
    base = wid * _PER_W
    gbase = pl.multiple_of(wid * _GROWS, 8)
    pltpu.sync_copy(idx_hbm.at[pl.ds(gbase, _GROWS)], idx_v)

    def fire_gathers(i, s):
        for j in range(_G):
            pltpu.async_copy(
                w_hbm.at[idx_v.at[i * _G + j]],
                rows_v.at[s].at[pl.ds(j * _IDX_MINOR, _IDX_MINOR)],
                gsem)

    def drain(sem, s):
        pltpu.make_async_copy(w_hbm.at[pl.ds(0, _C)], rows_v.at[s], sem).wait()

    def fire_store(i, s):
        row0 = pl.multiple_of(base + i * _C, 8)
        pltpu.async_copy(rows_v.at[s], out_hbm.at[pl.ds(row0, _C)], ssem)

    fire_gathers(0, 0)
    drain(gsem, 0)
    fire_store(0, 0)
    fire_gathers(1, 1)

    def body(i, carry):
        s = i % 2
        p = 1 - s
        drain(gsem, p)
        fire_store(i - 1, p)
        drain(ssem, s)
        fire_gathers(i, s)
        return carry

    lax.fori_loop(2, _NCHUNK, body, 0)

    last = _NCHUNK - 1
    sl = last % 2
    drain(gsem, sl)
    drain(ssem, 1 - sl)
    fire_store(last, sl)
    drain(ssem, sl)


def kernel(token_ids, weight):
    idx = token_ids.astype(jnp.int32).reshape(_NUM_ROWS // _IDX_MINOR, _IDX_MINOR)
    out = _emb_gather(weight, idx)
    return out.reshape(token_ids.shape + (_DIM,))


# speedup vs baseline: 1.9307x; 1.7396x over previous
import functools

import jax
import jax.numpy as jnp
from jax import lax
from jax.experimental import pallas as pl
from jax.experimental.pallas import tpu as pltpu
from jax.experimental.pallas import tpu_sc as plsc

_NUM_ROWS = 16384 * 50
_DIM = 32
_NW = 32
_PER_W = _NUM_ROWS // _NW
_IDX_MINOR = 128
_C = 1024
_G = _C // _IDX_MINOR
_NCHUNK = _PER_W // _C
_GROWS = _PER_W // _IDX_MINOR

_mesh = plsc.VectorSubcoreMesh(core_axis_name="c", subcore_axis_name="s")


@functools.partial(
    pl.kernel,
    mesh=_mesh,
    out_type=jax.ShapeDtypeStruct((_NUM_ROWS, _DIM), jnp.float32),
    scratch_types=[
        pltpu.VMEM((_GROWS, _IDX_MINOR), jnp.int32),
        pltpu.VMEM((2, _C, _DIM), jnp.float32),
        pltpu.SemaphoreType.DMA,
        pltpu.SemaphoreType.DMA,
    ],
    compiler_params=pltpu.CompilerParams(use_tc_tiling_on_sc=False),
)
def _emb_gather(w_hbm, idx_hbm, out_hbm, idx_v, rows_v, gsem, ssem):
    wid = lax.axis_index("s") * 2 + lax.axis_index("c")
    base = wid * _PER_W
    gbase = pl.multiple_of(wid * _GROWS, 8)
    pltpu.sync_copy(idx_hbm.at[pl.ds(gbase, _GROWS)], idx_v)

    def fire_gathers(i, s):
        for j in range(_G):
            pltpu.async_copy(
                w_hbm.at[idx_v.at[i * _G + j]],
                rows_v.at[s].at[pl.ds(j * _IDX_MINOR, _IDX_MINOR)],
                gsem)

    def drain(sem, s):
        pltpu.make_async_copy(w_hbm.at[pl.ds(0, _C)], rows_v.at[s], sem).wait()

    def fire_store(i, s):
        row0 = pl.multiple_of(base + i * _C, 8)
        pltpu.async_copy(rows_v.at[s], out_hbm.at[pl.ds(row0, _C)], ssem)

    fire_gathers(0, 0)
    drain(gsem, 0)
    fire_store(0, 0)
    fire_gathers(1, 1)

    def body(i, carry):
        s = i % 2
        p = 1 - s
        drain(gsem, p)
        fire_store(i - 1, p)
        drain(ssem, s)
        fire_gathers(i, s)
        return carry

    lax.fori_loop(2, _NCHUNK, body, 0)

    last = _NCHUNK - 1
    sl = last % 2
    drain(gsem, sl)
    drain(ssem, 1 - sl)
    fire_store(last, sl)
    drain(ssem, sl)


def kernel(token_ids, weight):
    b, p = token_ids.shape
    idx = token_ids.astype(jnp.int32).T.reshape(_NUM_ROWS // _IDX_MINOR, _IDX_MINOR)
    out = _emb_gather(weight, idx)
    return out.reshape(p, b, _DIM).transpose(1, 0, 2)
